# Initial kernel scaffold; baseline (speedup 1.0000x reference)
import jax
import jax.numpy as jnp
from jax.experimental import pallas as pl


def kernel(x, edge_index, W1, W2):
    raise NotImplementedError("write your pallas kernel here")


# speedup vs baseline: 7.1517x; 7.1517x over previous
import functools

import jax
import jax.numpy as jnp
from jax import lax
from jax.experimental import pallas as pl
from jax.experimental.pallas import tpu as pltpu
from jax.experimental.pallas import tpu_sc as plsc

_NC = 2
_NS = 16
_NW = _NC * _NS
_C = 128


def _make_spmm(NP, D, E_pad):
    PW = E_pad // _NW
    K = PW // _C
    RP = NP // _NS
    mesh = plsc.VectorSubcoreMesh(core_axis_name="c", subcore_axis_name="s")

    @functools.partial(
        pl.kernel,
        out_type=jax.ShapeDtypeStruct((_NC, NP, D), jnp.float32),
        mesh=mesh,
        scratch_types=[
            pltpu.VMEM((_C,), jnp.int32),
            pltpu.VMEM((_C,), jnp.int32),
            pltpu.VMEM((_C, D), jnp.float32),
            pltpu.VMEM((16, D), jnp.float32),
            pltpu.VMEM_SHARED((NP, D), jnp.float32),
            pltpu.SemaphoreType.DMA,
        ],
        compiler_params=pltpu.CompilerParams(use_tc_tiling_on_sc=False),
    )
    def spmm(x_hbm, col_hbm, row_hbm, out_hbm, colb, rowb, gbuf, zbuf, acc, sem):
        cid = lax.axis_index("c")
        sid = lax.axis_index("s")
        wid = sid * _NC + cid
        zv = jnp.zeros((16,), jnp.float32)
        for i in range(16):
            for j in range(D // 16):
                zbuf[i, pl.ds(j * 16, 16)] = zv
        for r in range(RP // 16):
            pltpu.sync_copy(zbuf, acc.at[pl.ds(sid * RP + r * 16, 16)])
        plsc.subcore_barrier()

        base = wid * PW

        def body(k, carry):
            off = base + k * _C
            pltpu.sync_copy(col_hbm.at[pl.ds(off, _C)], colb)
            pltpu.sync_copy(row_hbm.at[pl.ds(off, _C)], rowb)
            pltpu.async_copy(x_hbm.at[colb], gbuf, sem).wait()
            pltpu.sync_copy(gbuf, acc.at[rowb], add=True)
            return carry

        lax.fori_loop(0, K, body, 0)
        plsc.subcore_barrier()
        pltpu.sync_copy(acc.at[pl.ds(sid * RP, RP)],
                        out_hbm.at[cid, pl.ds(sid * RP, RP)])

    return spmm


def _mid_body(s1_ref, w1_ref, w2_ref, g_ref, invd_ref, *, IN):
    a = s1_ref[0] + s1_ref[1]
    deg = a[:, IN]
    invd = 1.0 / jnp.maximum(deg, 1.0)
    ax = a[:, :IN] * invd[:, None]
    h = jnp.maximum(jnp.dot(ax, w1_ref[...], preferred_element_type=jnp.float32), 0.0)
    g = jnp.dot(h, w2_ref[...], preferred_element_type=jnp.float32)
    g_ref[...] = g
    invd_ref[...] = jnp.broadcast_to(invd[:, None], invd_ref.shape)


def _fin_body(s2_ref, invd_ref, o_ref):
    o_ref[...] = (s2_ref[0] + s2_ref[1]) * invd_ref[...]


def kernel(x, edge_index, W1, W2):
    N, IN = x.shape
    H = W1.shape[1]
    E = edge_index.shape[1]
    D1 = IN + 16
    NP = 10240
    PW = -(-E // (_NW * _C)) * _C
    E_pad = PW * _NW

    row = edge_index[0].astype(jnp.int32)
    col = edge_index[1].astype(jnp.int32)
    padi = jnp.full((E_pad - E,), N, jnp.int32)
    colp = jnp.concatenate([col, padi])
    rowp = jnp.concatenate([row, padi])

    x_aug = jnp.zeros((NP, D1), jnp.float32)
    x_aug = x_aug.at[:N, :IN].set(x).at[:N, IN].set(1.0)

    spmm1 = _make_spmm(NP, D1, E_pad)
    spmm2 = _make_spmm(NP, IN, E_pad)

    s1 = spmm1(x_aug, colp, rowp)

    BN = 1024
    grid = (NP // BN,)
    g, invd = pl.pallas_call(
        functools.partial(_mid_body, IN=IN),
        grid=grid,
        in_specs=[
            pl.BlockSpec((_NC, BN, D1), lambda i: (0, i, 0)),
            pl.BlockSpec((IN, H), lambda i: (0, 0)),
            pl.BlockSpec((H, IN), lambda i: (0, 0)),
        ],
        out_specs=[
            pl.BlockSpec((BN, IN), lambda i: (i, 0)),
            pl.BlockSpec((BN, IN), lambda i: (i, 0)),
        ],
        out_shape=[
            jax.ShapeDtypeStruct((NP, IN), jnp.float32),
            jax.ShapeDtypeStruct((NP, IN), jnp.float32),
        ],
    )(s1, W1, W2)

    s2 = spmm2(g, colp, rowp)

    o = pl.pallas_call(
        _fin_body,
        grid=grid,
        in_specs=[
            pl.BlockSpec((_NC, BN, IN), lambda i: (0, i, 0)),
            pl.BlockSpec((BN, IN), lambda i: (i, 0)),
        ],
        out_specs=pl.BlockSpec((BN, IN), lambda i: (i, 0)),
        out_shape=jax.ShapeDtypeStruct((NP, IN), jnp.float32),
    )(s2, invd)
    return o[:N]
